# baseline (speedup 1.0000x reference)
import functools

import jax
import jax.numpy as jnp
from jax import lax
from jax.experimental import pallas as pl
from jax.experimental.pallas import tpu as pltpu
from jax.experimental.pallas import tpu_sc as plsc

NC, NS, L = 2, 16, 16
NW = NC * NS
B = 16384
D = 32
BPW = B // NW
CH = 128
NCH = BPW // CH

_mesh = plsc.VectorSubcoreMesh(core_axis_name="c", subcore_axis_name="s")


@functools.partial(
    pl.kernel,
    out_type=jax.ShapeDtypeStruct((B,), jnp.float32),
    mesh=_mesh,
    compiler_params=pltpu.CompilerParams(
        needs_layout_passes=False, use_tc_tiling_on_sc=False),
    scratch_types=[
        pltpu.VMEM((NCH, CH), jnp.int32),
        pltpu.VMEM((NCH, CH), jnp.int32),
        pltpu.VMEM((NCH, CH, D), jnp.float32),
        pltpu.VMEM((NCH, CH, D), jnp.float32),
        pltpu.VMEM((BPW,), jnp.float32),
        pltpu.SemaphoreType.DMA,
    ],
)
def _mf_sc(xu_hbm, xv_hbm, w_hbm, h_hbm, out_hbm,
           ui_v, vi_v, ur_v, vr_v, o_v, sem):
    wid = lax.axis_index("s") * NC + lax.axis_index("c")
    base = wid * BPW

    for k in range(NCH):
        pltpu.sync_copy(xu_hbm.at[pl.ds(base + k * CH, CH)], ui_v.at[k])
        pltpu.sync_copy(xv_hbm.at[pl.ds(base + k * CH, CH)], vi_v.at[k])

    copies = []
    for k in range(NCH):
        copies.append(pltpu.async_copy(w_hbm.at[ui_v.at[k]], ur_v.at[k], sem))
        copies.append(pltpu.async_copy(h_hbm.at[vi_v.at[k]], vr_v.at[k], sem))
    for c in copies:
        c.wait()

    lane = lax.iota(jnp.int32, L)

    def chunk_body(k, _):
        def group_body(g, _):
            acc = jnp.zeros((L,), jnp.float32)
            for j in range(L):
                r = g * L + j
                u0 = ur_v[k, r, pl.ds(0, L)]
                u1 = ur_v[k, r, pl.ds(L, L)]
                v0 = vr_v[k, r, pl.ds(0, L)]
                v1 = vr_v[k, r, pl.ds(L, L)]
                s = jnp.sum(u0 * v0 + u1 * v1)
                acc = jnp.where(lane == j, s, acc)
            o_v[pl.ds(k * CH + g * L, L)] = 1.0 / (1.0 + jnp.exp(-acc))
            return 0
        return lax.fori_loop(0, CH // L, group_body, 0)

    lax.fori_loop(0, NCH, chunk_body, 0)
    pltpu.sync_copy(o_v, out_hbm.at[pl.ds(base, BPW)])


def kernel(x, W, H):
    xi = x.astype(jnp.int32)
    return _mf_sc(xi[:, 0], xi[:, 1], W, H)
